# baseline (speedup 1.0000x reference)
import functools

import jax
import jax.numpy as jnp
from jax import lax
from jax.experimental import pallas as pl
from jax.experimental.pallas import tpu as pltpu
from jax.experimental.pallas import tpu_sc as plsc

_VOCAB = 1000000
_DIM = 16
_BATCH = 16384

_NC = 2
_NS = 16
_NW = _NC * _NS
_FULL_TILES = 7812
_STRIDE = 245
_CH = 12
_NCHUNK = 21
_SPAN = _CH * _NCHUNK
_LO_MAX = _FULL_TILES - _SPAN
_CW = _CH * 128
_CAP = 112

_mesh = plsc.VectorSubcoreMesh(core_axis_name="c", subcore_axis_name="s")


@functools.partial(
    pl.kernel,
    mesh=_mesh,
    out_type=jax.ShapeDtypeStruct((_BATCH, 128), jnp.float32),
    scratch_types=[
        pltpu.VMEM((_BATCH,), jnp.int32),
        pltpu.VMEM((_BATCH + 16,), jnp.int32),
        pltpu.VMEM((_BATCH + 16,), jnp.int32),
        pltpu.VMEM((2, _DIM, _CW), jnp.float32),
        pltpu.VMEM((_DIM, 144), jnp.float32),
        pltpu.VMEM((144,), jnp.int32),
        pltpu.VMEM((_DIM, 128), jnp.float32),
        pltpu.VMEM((_DIM, 128), jnp.float32),
        [pltpu.SemaphoreType.DMA] * 2,
        pltpu.SemaphoreType.DMA,
    ],
    compiler_params=pltpu.CompilerParams(needs_layout_passes=False),
)
def _gather_kernel(
    zt_hbm, tail_hbm, idx_hbm, out_hbm, idx_v, hv, hp, slab_v, ct, cpos,
    rows_v, tail_v, sems, ssem
):
    wid = lax.axis_index("s") * _NC + lax.axis_index("c")
    lo_tile = jnp.minimum(wid * _STRIDE, _LO_MAX)
    lo_col = lo_tile * 128
    is_last = wid == _NW - 1
    hi_col = jnp.where(is_last, _VOCAB, lo_col + _SPAN * 128)

    def fire(c, slot):
        col = pl.multiple_of(lo_col + c * _CW, 128)
        pltpu.async_copy(
            zt_hbm.at[:, pl.ds(col, _CW)], slab_v.at[slot], sems[slot]
        )

    fire(0, 0)
    fire(1, 1)

    pltpu.sync_copy(idx_hbm, idx_v)

    lanes = lax.iota(jnp.int32, 16)
    e_ids = lax.iota(jnp.int32, 16)

    def compact(i, n):
        vv = idx_v[pl.ds(i * 16, 16)]
        ps = i * 16 + lanes
        m = (vv >= lo_col) & (vv < hi_col)
        plsc.store_compressed(hv.at[pl.ds(n, 16)], vv, mask=m)
        plsc.store_compressed(hp.at[pl.ds(n, 16)], ps, mask=m)
        return n + plsc.all_reduce_population_count(m)[0]

    n_hits = lax.fori_loop(0, _BATCH // 16, compact, jnp.int32(0))
    n_grp = (n_hits + 15) // 16

    def flush(acc):
        def blk(b, _):
            @pl.when(b * 16 < acc)
            def _():
                for r in range(16):
                    row = plsc.load_gather(
                        ct, [e_ids, jnp.full((16,), b * 16 + r, jnp.int32)]
                    )
                    rows_v[r, pl.ds(0, _DIM)] = row
                pv = cpos[pl.ds(b * 16, 16)]
                pm = (b * 16 + lanes) < acc
                pv = jnp.where(pm, pv, jnp.int32(-1))
                pltpu.async_copy(
                    rows_v,
                    out_hbm.at[plsc.Indices(pv, ignored_value=-1)],
                    ssem,
                ).wait()
            return ()

        lax.fori_loop(0, 8, blk, ())

    def do_chunk(c_lo, c_hi, src_ref, acc):
        def grp(g, acc):
            vv = hv[pl.ds(g * 16, 16)]
            ok = (g * 16 + lanes) < n_hits
            m = (vv >= c_lo) & (vv < c_hi) & ok
            cnt = plsc.all_reduce_population_count(m)[0]

            def extract(acc):
                vloc = vv - c_lo
                pp = hp[pl.ds(g * 16, 16)]
                for e in range(_DIM):
                    vals = plsc.load_gather(
                        src_ref, [jnp.full((16,), e, jnp.int32), vloc],
                        mask=m,
                    )
                    plsc.store_compressed(ct.at[e, pl.ds(acc, 16)], vals, mask=m)
                plsc.store_compressed(cpos.at[pl.ds(acc, 16)], pp, mask=m)
                return acc + cnt

            def spill(acc):
                flush(acc)
                return jnp.int32(0)

            acc = lax.cond(acc > _CAP, spill, lambda a: a, acc)
            return lax.cond(cnt > 0, extract, lambda a: a, acc)

        return lax.fori_loop(0, n_grp, grp, acc)

    def chunk_loop(c, acc):
        for b in range(2):
            @pl.when(lax.rem(c, 2) == b)
            def _():
                pltpu.make_async_copy(
                    zt_hbm.at[:, pl.ds(0, _CW)], slab_v.at[b], sems[b]
                ).wait()

        c_lo = lo_col + c * _CW
        acc = lax.cond(
            lax.rem(c, 2) == 0,
            lambda a: do_chunk(c_lo, c_lo + _CW, slab_v.at[0], a),
            lambda a: do_chunk(c_lo, c_lo + _CW, slab_v.at[1], a),
            acc,
        )

        for b in range(2):
            @pl.when((lax.rem(c, 2) == b) & (c + 2 < _NCHUNK))
            def _():
                fire(c + 2, b)

        return acc

    acc = lax.fori_loop(0, _NCHUNK, chunk_loop, jnp.int32(0))

    @pl.when(is_last)
    def _():
        pltpu.sync_copy(tail_hbm, tail_v)

    acc = lax.cond(
        is_last,
        lambda a: do_chunk(
            jnp.int32(_VOCAB - 128), jnp.int32(_VOCAB), tail_v, a
        ),
        lambda a: a,
        acc,
    )
    flush(acc)


def kernel(Z, indices):
    idx = indices.astype(jnp.int32)
    tail = Z[_VOCAB - 128 :].T
    blob = _gather_kernel(Z.T, tail, idx)
    return blob[:, :_DIM]


# speedup vs baseline: 1.2793x; 1.2793x over previous
import functools

import jax
import jax.numpy as jnp
from jax import lax
from jax.experimental import pallas as pl
from jax.experimental.pallas import tpu as pltpu
from jax.experimental.pallas import tpu_sc as plsc

_VOCAB = 1000000
_DIM = 16
_BATCH = 16384

_NC = 2
_NS = 16
_NW = _NC * _NS
_BPW = _BATCH // _NW
_NBUF = 8

_mesh = plsc.VectorSubcoreMesh(core_axis_name="c", subcore_axis_name="s")


@functools.partial(
    pl.kernel,
    mesh=_mesh,
    out_type=jax.ShapeDtypeStruct((_DIM, _BATCH), jnp.float32),
    scratch_types=[
        pltpu.VMEM((_BPW + 16,), jnp.int32),
        pltpu.VMEM((_NBUF, _DIM, 128), jnp.float32),
        pltpu.VMEM((_BPW, _DIM), jnp.float32),
        pltpu.VMEM((_DIM, _BPW), jnp.float32),
        [pltpu.SemaphoreType.DMA] * _NBUF,
    ],
    compiler_params=pltpu.CompilerParams(needs_layout_passes=False),
)
def _gather_kernel(zt_hbm, idx_hbm, out_hbm, idx_v, win_v, rows_v, blk_v, sems):
    wid = lax.axis_index("s") * _NC + lax.axis_index("c")
    base = wid * _BPW
    pltpu.sync_copy(idx_hbm.at[pl.ds(base, _BPW)], idx_v.at[pl.ds(0, _BPW)])

    row_ids = lax.iota(jnp.int32, 16)

    def fire(j, slot):
        v = idx_v[pl.ds(j, 16)][0]
        col = pl.multiple_of((v // 128) * 128, 128)
        pltpu.async_copy(
            zt_hbm.at[pl.ds(0, 8), pl.ds(col, 128)],
            win_v.at[slot, pl.ds(0, 8)],
            sems[slot],
        )
        pltpu.async_copy(
            zt_hbm.at[pl.ds(8, 8), pl.ds(col, 128)],
            win_v.at[slot, pl.ds(8, 8)],
            sems[slot],
        )

    for b in range(_NBUF):
        fire(b, b)

    def group(g, _):
        for b in range(_NBUF):
            j = g * _NBUF + b
            pltpu.make_async_copy(
                zt_hbm.at[:, pl.ds(0, 128)], win_v.at[b], sems[b]
            ).wait()
            v = idx_v[pl.ds(j, 16)][0]
            vl = lax.rem(v, 128)
            row = plsc.load_gather(
                win_v.at[b], [row_ids, jnp.full((16,), vl, jnp.int32)]
            )
            rows_v[j, :] = row

            @pl.when(j + _NBUF < _BPW)
            def _():
                fire(j + _NBUF, b)

        return ()

    lax.fori_loop(0, _BPW // _NBUF, group, ())

    for e in range(_DIM):
        col_ids = jnp.full((16,), e, jnp.int32)
        for c in range(_BPW // 16):
            vals = plsc.load_gather(rows_v, [c * 16 + row_ids, col_ids])
            blk_v[e, pl.ds(c * 16, 16)] = vals
    pltpu.sync_copy(blk_v, out_hbm.at[:, pl.ds(base, _BPW)])


def kernel(Z, indices):
    idx = indices.astype(jnp.int32)
    out_t = _gather_kernel(Z.T, idx)
    return out_t.T
